# baseline (speedup 1.0000x reference)
import functools

import jax
import jax.numpy as jnp
from jax import lax
from jax.experimental import pallas as pl
from jax.experimental.pallas import tpu as pltpu
from jax.experimental.pallas import tpu_sc as plsc

N_NODES = 10000
N_EDGES = 320000
D = 128
N_GRAPHS = 64
N_CLASSES = 10

NC, NS = 2, 16
NW = NC * NS
E_PER_W = N_EDGES // NW
EB = 128
N_FULL = E_PER_W // EB
TAIL = E_PER_W - N_FULL * EB
DEG_CHUNK = 2000
AGG_ROWS = 10112
ROWS_PER_SUB = AGG_ROWS // NS

_mesh = plsc.VectorSubcoreMesh(core_axis_name="c", subcore_axis_name="s")


def _wid():
    return lax.axis_index("s") * NC + lax.axis_index("c")


def _sigmoid(v):
    return 1.0 / (1.0 + jnp.exp(-v))


def _deg_body(col_hbm, pv_hbm, out_hbm, degpart, colbuf, pvbuf):
    w = _wid()
    def zero(i, _):
        degpart[pl.ds(i * 16, 16)] = jnp.zeros((16,), jnp.float32)
        return 0
    lax.fori_loop(0, N_NODES // 16, zero, 0)
    base = w * E_PER_W
    def chunk(cidx, _):
        off = base + cidx * DEG_CHUNK
        pltpu.sync_copy(col_hbm.at[pl.ds(off, DEG_CHUNK)], colbuf)
        pltpu.sync_copy(pv_hbm.at[pl.ds(off, DEG_CHUNK)], pvbuf)
        def grp(k, _):
            c16 = colbuf[pl.ds(k * 16, 16)]
            p16 = _sigmoid(pvbuf[pl.ds(k * 16, 16)])
            plsc.addupdate_scatter(degpart, [c16], p16)
            return 0
        lax.fori_loop(0, DEG_CHUNK // 16, grp, 0)
        return 0
    lax.fori_loop(0, E_PER_W // DEG_CHUNK, chunk, 0)
    pltpu.sync_copy(degpart, out_hbm.at[w, 0])


_deg_kernel = pl.kernel(
    _deg_body,
    out_type=jax.ShapeDtypeStruct((NW, 1, N_NODES), jnp.float32),
    mesh=_mesh,
    compiler_params=pltpu.CompilerParams(needs_layout_passes=False),
    scratch_types=[
        pltpu.VMEM((N_NODES,), jnp.float32),
        pltpu.VMEM((DEG_CHUNK,), jnp.int32),
        pltpu.VMEM((DEG_CHUNK,), jnp.float32),
    ],
)


def _dis_body(parts_ref, dis_ref, sl_ref):
    deg = 1.0 + jnp.sum(parts_ref[...], axis=0, keepdims=True)
    dis_ref[...] = lax.rsqrt(deg)
    sl_ref[...] = 1.0 / deg


def _dis_call(parts):
    return pl.pallas_call(
        _dis_body,
        out_shape=[
            jax.ShapeDtypeStruct((1, N_NODES), jnp.float32),
            jax.ShapeDtypeStruct((1, N_NODES), jnp.float32),
        ],
    )(parts)


def _w_body(row_hbm, col_hbm, pv_hbm, dis_hbm, w_hbm,
            disbuf, rowbuf, colbuf, pvbuf, wbuf):
    w = _wid()
    pltpu.sync_copy(dis_hbm, disbuf)
    base = w * E_PER_W
    def chunk(cidx, _):
        off = base + cidx * DEG_CHUNK
        pltpu.sync_copy(row_hbm.at[pl.ds(off, DEG_CHUNK)], rowbuf)
        pltpu.sync_copy(col_hbm.at[pl.ds(off, DEG_CHUNK)], colbuf)
        pltpu.sync_copy(pv_hbm.at[pl.ds(off, DEG_CHUNK)], pvbuf)
        def grp(k, _):
            r16 = rowbuf[pl.ds(k * 16, 16)]
            c16 = colbuf[pl.ds(k * 16, 16)]
            p16 = _sigmoid(pvbuf[pl.ds(k * 16, 16)])
            dr = plsc.load_gather(disbuf, [r16])
            dc = plsc.load_gather(disbuf, [c16])
            wbuf[pl.ds(k * 16, 16)] = dr * p16 * dc
            return 0
        lax.fori_loop(0, DEG_CHUNK // 16, grp, 0)
        pltpu.sync_copy(wbuf, w_hbm.at[pl.ds(off, DEG_CHUNK)])
        return 0
    lax.fori_loop(0, E_PER_W // DEG_CHUNK, chunk, 0)


_w_kernel = pl.kernel(
    _w_body,
    out_type=jax.ShapeDtypeStruct((N_EDGES,), jnp.float32),
    mesh=_mesh,
    compiler_params=pltpu.CompilerParams(needs_layout_passes=False),
    scratch_types=[
        pltpu.VMEM((N_NODES,), jnp.float32),
        pltpu.VMEM((DEG_CHUNK,), jnp.int32),
        pltpu.VMEM((DEG_CHUNK,), jnp.int32),
        pltpu.VMEM((DEG_CHUNK,), jnp.float32),
        pltpu.VMEM((DEG_CHUNK,), jnp.float32),
    ],
)


def _splat(v):
    return jnp.full((16,), v, jnp.int32)


def _scale_rows_fori(buf, wbuf, n):
    def f(g, _):
        k0 = g * 16
        wv = wbuf[pl.ds(k0, 16)]
        for kk in range(16):
            uv = lax.gather(
                wv, jnp.full((16, 1), kk, jnp.int32),
                lax.GatherDimensionNumbers(offset_dims=(),
                                           collapsed_slice_dims=(0,),
                                           start_index_map=(0,)),
                (1,), mode=lax.GatherScatterMode.PROMISE_IN_BOUNDS)
            k = k0 + kk
            for j in range(8):
                buf[k, j * 16:(j + 1) * 16] = uv * buf[k, j * 16:(j + 1) * 16]
        return 0
    lax.fori_loop(0, n // 16, f, 0)


def _agg_body(z_hbm, row_hbm, col_hbm, w_hbm, out_hbm,
              acc, gbuf0, gbuf1, gbuf2, rowbuf0, rowbuf1,
              colbuf0, colbuf1, colbuf2, wbuf0, wbuf1, colbuf_t,
              zsem, gsem, rsem0, rsem1, csem0, csem1, csem2, wsem0, wsem1,
              ssem0, ssem1, ssem2):
    c = lax.axis_index("c")
    s = lax.axis_index("s")
    wid = s * NC + c
    base = wid * E_PER_W
    gbuf = (gbuf0, gbuf1, gbuf2)
    rowbuf = (rowbuf0, rowbuf1)
    colbuf = (colbuf0, colbuf1, colbuf2)
    wbuf = (wbuf0, wbuf1)
    rsem = (rsem0, rsem1)
    csem = (csem0, csem1, csem2)
    wsem = (wsem0, wsem1)
    ssem = (ssem0, ssem1, ssem2)

    def fire_row(i, b):
        pltpu.async_copy(row_hbm.at[pl.ds(base + i * EB, EB)], rowbuf[b],
                         rsem[b])

    def fire_col(i, t):
        pltpu.async_copy(col_hbm.at[pl.ds(base + i * EB, EB)], colbuf[t],
                         csem[t])

    def fire_w(i, b):
        pltpu.async_copy(w_hbm.at[pl.ds(base + i * EB, EB)], wbuf[b], wsem[b])

    def wait2(dst, sem, n=EB):
        pltpu.make_async_copy(z_hbm.at[pl.ds(0, n)], dst, sem).wait()

    def wait_i(dst, sem, n=EB):
        pltpu.make_async_copy(row_hbm.at[pl.ds(0, n)], dst, sem).wait()

    def wait_f(dst, sem, n=EB):
        pltpu.make_async_copy(w_hbm.at[pl.ds(0, n)], dst, sem).wait()

    fire_row(0, 0)
    fire_row(1, 1)
    fire_col(0, 0)
    fire_w(0, 0)
    def zz(i, _):
        for j in range(8):
            gbuf1[i, j * 16:(j + 1) * 16] = jnp.zeros((16,), jnp.float32)
        return 0
    zrows = 0
    lax.fori_loop(0, EB, zz, 0)
    nbase = s * ROWS_PER_SUB
    for t in range(ROWS_PER_SUB // EB):
        pltpu.async_copy(gbuf1, acc.at[pl.ds(nbase + t * EB, EB)], zsem)
        zrows += EB
    rem = ROWS_PER_SUB - zrows
    pltpu.async_copy(gbuf1.at[pl.ds(0, rem)],
                     acc.at[pl.ds(nbase + zrows, rem)], zsem)
    wait_i(rowbuf0, rsem[0])
    pltpu.async_copy(z_hbm.at[rowbuf0], gbuf0, gsem)
    for t in range(ROWS_PER_SUB // EB):
        pltpu.make_async_copy(z_hbm.at[pl.ds(0, EB)], gbuf1, zsem).wait()
    pltpu.make_async_copy(z_hbm.at[pl.ds(0, rem)], gbuf1.at[pl.ds(0, rem)],
                          zsem).wait()
    plsc.subcore_barrier()

    def step(i, t, b2):
        tn = (t + 1) % 3
        wait2(gbuf[t], gsem)

        @pl.when(i >= 2)
        def _():
            wait2(gbuf[tn], ssem[tn])

        @pl.when(i + 1 < N_FULL)
        def _():
            wait_i(rowbuf[1 - b2], rsem[1 - b2])
            pltpu.async_copy(z_hbm.at[rowbuf[1 - b2]], gbuf[tn], gsem)
            fire_col(i + 1, tn)
            fire_w(i + 1, 1 - b2)

        @pl.when(i + 2 < N_FULL)
        def _():
            fire_row(i + 2, b2)

        wait_f(wbuf[b2], wsem[b2])
        _scale_rows_fori(gbuf[t], wbuf[b2], EB)
        wait_i(colbuf[t], csem[t])
        pltpu.async_copy(gbuf[t], acc.at[colbuf[t]], ssem[t], add=True)

    def outer(g, _):
        for u in range(6):
            i = g * 6 + u
            step(i, u % 3, u % 2)
        return 0
    lax.fori_loop(0, N_FULL // 6, outer, 0)
    wait2(gbuf[(N_FULL - 2) % 3], ssem[(N_FULL - 2) % 3])
    wait2(gbuf[(N_FULL - 1) % 3], ssem[(N_FULL - 1) % 3])

    toff = base + N_FULL * EB
    pltpu.async_copy(col_hbm.at[pl.ds(toff, TAIL)], colbuf_t, csem[0])
    wt = wbuf0.at[pl.ds(0, TAIL)]
    pltpu.async_copy(w_hbm.at[pl.ds(toff, TAIL)], wt, wsem[0])
    rt = rowbuf0.at[pl.ds(0, TAIL)]
    pltpu.async_copy(row_hbm.at[pl.ds(toff, TAIL)], rt, rsem[0])
    wait_i(rt, rsem[0], TAIL)
    gt = gbuf0.at[pl.ds(0, TAIL)]
    pltpu.async_copy(z_hbm.at[rt], gt, gsem)
    wait2(gt, gsem, TAIL)
    wait_f(wt, wsem[0], TAIL)
    _scale_rows_fori(gbuf0, wbuf0, TAIL)
    wait_i(colbuf_t, csem[0], TAIL)
    pltpu.sync_copy(gt, acc.at[colbuf_t], add=True)

    plsc.subcore_barrier()
    pltpu.sync_copy(acc.at[pl.ds(s * ROWS_PER_SUB, ROWS_PER_SUB)],
                    out_hbm.at[c, pl.ds(s * ROWS_PER_SUB, ROWS_PER_SUB)])


_agg_kernel = pl.kernel(
    _agg_body,
    out_type=jax.ShapeDtypeStruct((NC, AGG_ROWS, D), jnp.float32),
    mesh=_mesh,
    compiler_params=pltpu.CompilerParams(needs_layout_passes=False),
    scratch_types=[
        pltpu.VMEM_SHARED((AGG_ROWS, D), jnp.float32),
        pltpu.VMEM((EB, D), jnp.float32),
        pltpu.VMEM((EB, D), jnp.float32),
        pltpu.VMEM((EB, D), jnp.float32),
        pltpu.VMEM((EB,), jnp.int32),
        pltpu.VMEM((EB,), jnp.int32),
        pltpu.VMEM((EB,), jnp.int32),
        pltpu.VMEM((EB,), jnp.int32),
        pltpu.VMEM((EB,), jnp.int32),
        pltpu.VMEM((EB,), jnp.float32),
        pltpu.VMEM((EB,), jnp.float32),
        pltpu.VMEM((TAIL,), jnp.int32),
    ] + [pltpu.SemaphoreType.DMA] * 12,
)


RB = 2000


def _mm1_body(x_ref, parts_ref, w_ref, o_ref, dis_ref, sl_ref):
    o_ref[...] = lax.dot_general(
        x_ref[...], w_ref[...], (((1,), (1,)), ((), ())),
        preferred_element_type=jnp.float32)

    @pl.when(pl.program_id(0) == 0)
    def _():
        deg = 1.0 + jnp.sum(parts_ref[...], axis=0, keepdims=True)
        dis_ref[...] = lax.rsqrt(deg)
        sl_ref[...] = 1.0 / deg


def _mm1_call(x, parts, W):
    return pl.pallas_call(
        _mm1_body,
        grid=(N_NODES // RB,),
        in_specs=[
            pl.BlockSpec((RB, D), lambda i: (i, 0)),
            pl.BlockSpec((NW, N_NODES), lambda i: (0, 0)),
            pl.BlockSpec((D, D), lambda i: (0, 0)),
        ],
        out_specs=[
            pl.BlockSpec((RB, D), lambda i: (i, 0)),
            pl.BlockSpec((1, N_NODES), lambda i: (0, 0)),
            pl.BlockSpec((1, N_NODES), lambda i: (0, 0)),
        ],
        out_shape=[
            jax.ShapeDtypeStruct((N_NODES, D), jnp.float32),
            jax.ShapeDtypeStruct((1, N_NODES), jnp.float32),
            jax.ShapeDtypeStruct((1, N_NODES), jnp.float32),
        ],
    )(x, parts, W)


def _layer_body(p_ref, z_ref, sl_ref, b_ref, w_ref, o_ref, *, relu):
    x = p_ref[0] + p_ref[1] + sl_ref[...] * z_ref[...] + b_ref[...]
    if relu:
        x = jnp.maximum(x, 0.0)
    o_ref[...] = lax.dot_general(
        x, w_ref[...], (((1,), (1,)), ((), ())),
        preferred_element_type=jnp.float32)


def _layer_call(p, z, sl, b, W, relu):
    return pl.pallas_call(
        functools.partial(_layer_body, relu=relu),
        grid=(N_NODES // RB,),
        in_specs=[
            pl.BlockSpec((NC, RB, D), lambda i: (0, i, 0)),
            pl.BlockSpec((RB, D), lambda i: (i, 0)),
            pl.BlockSpec((RB, 1), lambda i: (i, 0)),
            pl.BlockSpec((1, D), lambda i: (0, 0)),
            pl.BlockSpec((D, D), lambda i: (0, 0)),
        ],
        out_specs=pl.BlockSpec((RB, D), lambda i: (i, 0)),
        out_shape=jax.ShapeDtypeStruct((N_NODES, D), jnp.float32),
    )(p, z, sl, b, W)


def _final_body(p_ref, z_ref, sl_ref, b_ref, batch_ref, wo_ref, bo_ref,
                o_ref, acc, cnt):
    i = pl.program_id(0)

    @pl.when(i == 0)
    def _():
        acc[...] = jnp.zeros_like(acc)
        cnt[...] = jnp.zeros_like(cnt)

    h = p_ref[0] + p_ref[1] + sl_ref[...] * z_ref[...] + b_ref[...]
    bt = batch_ref[0]
    gids = lax.broadcasted_iota(jnp.int32, (N_GRAPHS, RB), 0)
    onehot = jnp.where(bt == gids, 1.0, 0.0)
    acc[...] += lax.dot_general(
        onehot, h, (((1,), (0,)), ((), ())), preferred_element_type=jnp.float32)
    cnt[...] += jnp.broadcast_to(
        jnp.sum(onehot, axis=1, keepdims=True), (N_GRAPHS, D))

    @pl.when(i == N_NODES // RB - 1)
    def _():
        pooled = acc[...] / jnp.maximum(cnt[...], 1.0)
        o_ref[...] = lax.dot_general(
            pooled, wo_ref[...], (((1,), (1,)), ((), ())),
            preferred_element_type=jnp.float32) + bo_ref[...]


def _final_call(p, z, sl, b, batch4, Wo, bo):
    return pl.pallas_call(
        _final_body,
        grid=(N_NODES // RB,),
        in_specs=[
            pl.BlockSpec((NC, RB, D), lambda i: (0, i, 0)),
            pl.BlockSpec((RB, D), lambda i: (i, 0)),
            pl.BlockSpec((RB, 1), lambda i: (i, 0)),
            pl.BlockSpec((1, D), lambda i: (0, 0)),
            pl.BlockSpec((1, 1, RB), lambda i: (i, 0, 0)),
            pl.BlockSpec((N_CLASSES, D), lambda i: (0, 0)),
            pl.BlockSpec((1, N_CLASSES), lambda i: (0, 0)),
        ],
        out_specs=pl.BlockSpec((N_GRAPHS, N_CLASSES), lambda i: (0, 0)),
        out_shape=jax.ShapeDtypeStruct((N_GRAPHS, N_CLASSES), jnp.float32),
        scratch_shapes=[
            pltpu.VMEM((N_GRAPHS, D), jnp.float32),
            pltpu.VMEM((N_GRAPHS, D), jnp.float32),
        ],
    )(p, z, sl, b, batch4, Wo, bo)


def kernel(x, edge_index, batch, P_vec, W1, b1, W2, b2, W3, b3, Wo, bo):
    row = edge_index[0]
    col = edge_index[1]
    batch4 = batch.reshape(N_NODES // RB, 1, RB)

    deg_parts = _deg_kernel(col, P_vec).reshape(NW, N_NODES)
    z1, dis, sl = _mm1_call(x, deg_parts, W1)
    dis = dis.reshape(N_NODES)
    sl = sl.reshape(N_NODES, 1)
    w = _w_kernel(row, col, P_vec, dis)

    p1 = _agg_kernel(z1, row, col, w)
    z2 = _layer_call(p1, z1, sl, b1.reshape(1, D), W2, relu=True)
    p2 = _agg_kernel(z2, row, col, w)
    z3 = _layer_call(p2, z2, sl, b2.reshape(1, D), W3, relu=True)
    p3 = _agg_kernel(z3, row, col, w)
    return _final_call(p3, z3, sl, b3.reshape(1, D), batch4, Wo,
                       bo.reshape(1, N_CLASSES))


# speedup vs baseline: 1.0111x; 1.0111x over previous
import functools

import jax
import jax.numpy as jnp
from jax import lax
from jax.experimental import pallas as pl
from jax.experimental.pallas import tpu as pltpu
from jax.experimental.pallas import tpu_sc as plsc

N_NODES = 10000
N_EDGES = 320000
D = 128
N_GRAPHS = 64
N_CLASSES = 10

NC, NS = 2, 16
NW = NC * NS
E_PER_W = N_EDGES // NW
EB = 128
N_FULL = E_PER_W // EB
TAIL = E_PER_W - N_FULL * EB
DEG_CHUNK = 2000
AGG_ROWS = 10112
ROWS_PER_SUB = AGG_ROWS // NS

_mesh = plsc.VectorSubcoreMesh(core_axis_name="c", subcore_axis_name="s")


def _wid():
    return lax.axis_index("s") * NC + lax.axis_index("c")


def _sigmoid(v):
    return 1.0 / (1.0 + jnp.exp(-v))


def _deg_body(col_hbm, pv_hbm, out_hbm, degpart, colbuf, pvbuf):
    w = _wid()
    def zero(i, _):
        degpart[pl.ds(i * 16, 16)] = jnp.zeros((16,), jnp.float32)
        return 0
    lax.fori_loop(0, N_NODES // 16, zero, 0)
    base = w * E_PER_W
    def chunk(cidx, _):
        off = base + cidx * DEG_CHUNK
        pltpu.sync_copy(col_hbm.at[pl.ds(off, DEG_CHUNK)], colbuf)
        pltpu.sync_copy(pv_hbm.at[pl.ds(off, DEG_CHUNK)], pvbuf)
        def grp(k, _):
            c16 = colbuf[pl.ds(k * 16, 16)]
            p16 = _sigmoid(pvbuf[pl.ds(k * 16, 16)])
            plsc.addupdate_scatter(degpart, [c16], p16)
            return 0
        lax.fori_loop(0, DEG_CHUNK // 16, grp, 0)
        return 0
    lax.fori_loop(0, E_PER_W // DEG_CHUNK, chunk, 0)
    pltpu.sync_copy(degpart, out_hbm.at[w, 0])


_deg_kernel = pl.kernel(
    _deg_body,
    out_type=jax.ShapeDtypeStruct((NW, 1, N_NODES), jnp.float32),
    mesh=_mesh,
    compiler_params=pltpu.CompilerParams(needs_layout_passes=False),
    scratch_types=[
        pltpu.VMEM((N_NODES,), jnp.float32),
        pltpu.VMEM((DEG_CHUNK,), jnp.int32),
        pltpu.VMEM((DEG_CHUNK,), jnp.float32),
    ],
)


def _dis_body(parts_ref, dis_ref, sl_ref):
    deg = 1.0 + jnp.sum(parts_ref[...], axis=0, keepdims=True)
    dis_ref[...] = lax.rsqrt(deg)
    sl_ref[...] = 1.0 / deg


def _dis_call(parts):
    return pl.pallas_call(
        _dis_body,
        out_shape=[
            jax.ShapeDtypeStruct((1, N_NODES), jnp.float32),
            jax.ShapeDtypeStruct((1, N_NODES), jnp.float32),
        ],
    )(parts)


def _w_body(row_hbm, col_hbm, pv_hbm, dis_hbm, w_hbm,
            disbuf, rowbuf, colbuf, pvbuf, wbuf):
    w = _wid()
    pltpu.sync_copy(dis_hbm, disbuf)
    base = w * E_PER_W
    def chunk(cidx, _):
        off = base + cidx * DEG_CHUNK
        pltpu.sync_copy(row_hbm.at[pl.ds(off, DEG_CHUNK)], rowbuf)
        pltpu.sync_copy(col_hbm.at[pl.ds(off, DEG_CHUNK)], colbuf)
        pltpu.sync_copy(pv_hbm.at[pl.ds(off, DEG_CHUNK)], pvbuf)
        def grp(k, _):
            r16 = rowbuf[pl.ds(k * 16, 16)]
            c16 = colbuf[pl.ds(k * 16, 16)]
            p16 = _sigmoid(pvbuf[pl.ds(k * 16, 16)])
            dr = plsc.load_gather(disbuf, [r16])
            dc = plsc.load_gather(disbuf, [c16])
            wbuf[pl.ds(k * 16, 16)] = dr * p16 * dc
            return 0
        lax.fori_loop(0, DEG_CHUNK // 16, grp, 0)
        pltpu.sync_copy(wbuf, w_hbm.at[pl.ds(off, DEG_CHUNK)])
        return 0
    lax.fori_loop(0, E_PER_W // DEG_CHUNK, chunk, 0)


_w_kernel = pl.kernel(
    _w_body,
    out_type=jax.ShapeDtypeStruct((N_EDGES,), jnp.float32),
    mesh=_mesh,
    compiler_params=pltpu.CompilerParams(needs_layout_passes=False),
    scratch_types=[
        pltpu.VMEM((N_NODES,), jnp.float32),
        pltpu.VMEM((DEG_CHUNK,), jnp.int32),
        pltpu.VMEM((DEG_CHUNK,), jnp.int32),
        pltpu.VMEM((DEG_CHUNK,), jnp.float32),
        pltpu.VMEM((DEG_CHUNK,), jnp.float32),
    ],
)


def _splat(v):
    return jnp.full((16,), v, jnp.int32)


def _scale_rows_fori(buf, wbuf, n):
    def f(g, _):
        k0 = g * 16
        wv = wbuf[pl.ds(k0, 16)]
        for kk in range(16):
            uv = lax.gather(
                wv, jnp.full((16, 1), kk, jnp.int32),
                lax.GatherDimensionNumbers(offset_dims=(),
                                           collapsed_slice_dims=(0,),
                                           start_index_map=(0,)),
                (1,), mode=lax.GatherScatterMode.PROMISE_IN_BOUNDS)
            k = k0 + kk
            for j in range(8):
                buf[k, j * 16:(j + 1) * 16] = uv * buf[k, j * 16:(j + 1) * 16]
        return 0
    lax.fori_loop(0, n // 16, f, 0)


def _agg_body(z_hbm, row_hbm, col_hbm, w_hbm, out_hbm,
              acc, gbuf0, gbuf1, gbuf2, rowbuf0, rowbuf1,
              colbuf0, colbuf1, colbuf2, wbuf0, wbuf1, colbuf_t,
              zsem, gsem, rsem0, rsem1, csem0, csem1, csem2, wsem0, wsem1,
              ssem0, ssem1, ssem2):
    c = lax.axis_index("c")
    s = lax.axis_index("s")
    wid = s * NC + c
    base = wid * E_PER_W
    gbuf = (gbuf0, gbuf1, gbuf2)
    rowbuf = (rowbuf0, rowbuf1)
    colbuf = (colbuf0, colbuf1, colbuf2)
    wbuf = (wbuf0, wbuf1)
    rsem = (rsem0, rsem1)
    csem = (csem0, csem1, csem2)
    wsem = (wsem0, wsem1)
    ssem = (ssem0, ssem1, ssem2)

    def fire_row(i, b):
        pltpu.async_copy(row_hbm.at[pl.ds(base + i * EB, EB)], rowbuf[b],
                         rsem[b])

    def fire_col(i, t):
        pltpu.async_copy(col_hbm.at[pl.ds(base + i * EB, EB)], colbuf[t],
                         csem[t])

    def fire_w(i, b):
        pltpu.async_copy(w_hbm.at[pl.ds(base + i * EB, EB)], wbuf[b], wsem[b])

    def wait2(dst, sem, n=EB):
        pltpu.make_async_copy(z_hbm.at[pl.ds(0, n)], dst, sem).wait()

    def wait_i(dst, sem, n=EB):
        pltpu.make_async_copy(row_hbm.at[pl.ds(0, n)], dst, sem).wait()

    def wait_f(dst, sem, n=EB):
        pltpu.make_async_copy(w_hbm.at[pl.ds(0, n)], dst, sem).wait()

    fire_row(0, 0)
    fire_row(1, 1)
    fire_col(0, 0)
    fire_w(0, 0)
    def zz(i, _):
        for j in range(8):
            gbuf1[i, j * 16:(j + 1) * 16] = jnp.zeros((16,), jnp.float32)
        return 0
    zrows = 0
    lax.fori_loop(0, EB, zz, 0)
    nbase = s * ROWS_PER_SUB
    for t in range(ROWS_PER_SUB // EB):
        pltpu.async_copy(gbuf1, acc.at[pl.ds(nbase + t * EB, EB)], zsem)
        zrows += EB
    rem = ROWS_PER_SUB - zrows
    pltpu.async_copy(gbuf1.at[pl.ds(0, rem)],
                     acc.at[pl.ds(nbase + zrows, rem)], zsem)
    wait_i(rowbuf0, rsem[0])
    pltpu.async_copy(z_hbm.at[rowbuf0], gbuf0, gsem)
    for t in range(ROWS_PER_SUB // EB):
        pltpu.make_async_copy(z_hbm.at[pl.ds(0, EB)], gbuf1, zsem).wait()
    pltpu.make_async_copy(z_hbm.at[pl.ds(0, rem)], gbuf1.at[pl.ds(0, rem)],
                          zsem).wait()
    plsc.subcore_barrier()

    def step(i, t, b2):
        tn = (t + 1) % 3
        wait2(gbuf[t], gsem)

        @pl.when(i >= 2)
        def _():
            wait2(gbuf[tn], ssem[tn])

        @pl.when(i + 1 < N_FULL)
        def _():
            wait_i(rowbuf[1 - b2], rsem[1 - b2])
            pltpu.async_copy(z_hbm.at[rowbuf[1 - b2]], gbuf[tn], gsem)
            fire_col(i + 1, tn)
            fire_w(i + 1, 1 - b2)

        @pl.when(i + 2 < N_FULL)
        def _():
            fire_row(i + 2, b2)

        wait_f(wbuf[b2], wsem[b2])
        _scale_rows_fori(gbuf[t], wbuf[b2], EB)
        wait_i(colbuf[t], csem[t])
        pltpu.async_copy(gbuf[t], acc.at[colbuf[t]], ssem[t], add=True)

    def outer(g, _):
        for u in range(6):
            i = g * 6 + u
            step(i, u % 3, u % 2)
        return 0
    lax.fori_loop(0, N_FULL // 6, outer, 0)
    wait2(gbuf[(N_FULL - 2) % 3], ssem[(N_FULL - 2) % 3])
    wait2(gbuf[(N_FULL - 1) % 3], ssem[(N_FULL - 1) % 3])

    toff = base + N_FULL * EB
    pltpu.async_copy(col_hbm.at[pl.ds(toff, TAIL)], colbuf_t, csem[0])
    wt = wbuf0.at[pl.ds(0, TAIL)]
    pltpu.async_copy(w_hbm.at[pl.ds(toff, TAIL)], wt, wsem[0])
    rt = rowbuf0.at[pl.ds(0, TAIL)]
    pltpu.async_copy(row_hbm.at[pl.ds(toff, TAIL)], rt, rsem[0])
    wait_i(rt, rsem[0], TAIL)
    gt = gbuf0.at[pl.ds(0, TAIL)]
    pltpu.async_copy(z_hbm.at[rt], gt, gsem)
    wait2(gt, gsem, TAIL)
    wait_f(wt, wsem[0], TAIL)
    _scale_rows_fori(gbuf0, wbuf0, TAIL)
    wait_i(colbuf_t, csem[0], TAIL)
    pltpu.sync_copy(gt, acc.at[colbuf_t], add=True)

    plsc.subcore_barrier()
    pltpu.sync_copy(acc.at[pl.ds(s * ROWS_PER_SUB, ROWS_PER_SUB)],
                    out_hbm.at[c, pl.ds(s * ROWS_PER_SUB, ROWS_PER_SUB)])


_agg_kernel = pl.kernel(
    _agg_body,
    out_type=jax.ShapeDtypeStruct((NC, AGG_ROWS, D), jnp.float32),
    mesh=_mesh,
    compiler_params=pltpu.CompilerParams(needs_layout_passes=False),
    scratch_types=[
        pltpu.VMEM_SHARED((AGG_ROWS, D), jnp.float32),
        pltpu.VMEM((EB, D), jnp.float32),
        pltpu.VMEM((EB, D), jnp.float32),
        pltpu.VMEM((EB, D), jnp.float32),
        pltpu.VMEM((EB,), jnp.int32),
        pltpu.VMEM((EB,), jnp.int32),
        pltpu.VMEM((EB,), jnp.int32),
        pltpu.VMEM((EB,), jnp.int32),
        pltpu.VMEM((EB,), jnp.int32),
        pltpu.VMEM((EB,), jnp.float32),
        pltpu.VMEM((EB,), jnp.float32),
        pltpu.VMEM((TAIL,), jnp.int32),
    ] + [pltpu.SemaphoreType.DMA] * 12,
)


RB = 2000


def _mm1_body(x_ref, w_ref, o_ref):
    o_ref[...] = lax.dot_general(
        x_ref[...], w_ref[...], (((1,), (1,)), ((), ())),
        preferred_element_type=jnp.float32)


def _mm1_call(x, W):
    return pl.pallas_call(
        _mm1_body,
        grid=(N_NODES // RB,),
        in_specs=[
            pl.BlockSpec((RB, D), lambda i: (i, 0)),
            pl.BlockSpec((D, D), lambda i: (0, 0)),
        ],
        out_specs=pl.BlockSpec((RB, D), lambda i: (i, 0)),
        out_shape=jax.ShapeDtypeStruct((N_NODES, D), jnp.float32),
    )(x, W)


def _layer_body(p_ref, z_ref, sl_ref, b_ref, w_ref, o_ref, *, relu):
    x = p_ref[0] + p_ref[1] + sl_ref[...] * z_ref[...] + b_ref[...]
    if relu:
        x = jnp.maximum(x, 0.0)
    o_ref[...] = lax.dot_general(
        x, w_ref[...], (((1,), (1,)), ((), ())),
        preferred_element_type=jnp.float32)


def _layer_call(p, z, sl, b, W, relu):
    return pl.pallas_call(
        functools.partial(_layer_body, relu=relu),
        grid=(N_NODES // RB,),
        in_specs=[
            pl.BlockSpec((NC, RB, D), lambda i: (0, i, 0)),
            pl.BlockSpec((RB, D), lambda i: (i, 0)),
            pl.BlockSpec((RB, 1), lambda i: (i, 0)),
            pl.BlockSpec((1, D), lambda i: (0, 0)),
            pl.BlockSpec((D, D), lambda i: (0, 0)),
        ],
        out_specs=pl.BlockSpec((RB, D), lambda i: (i, 0)),
        out_shape=jax.ShapeDtypeStruct((N_NODES, D), jnp.float32),
    )(p, z, sl, b, W)


def _final_body(p_ref, z_ref, sl_ref, b_ref, batch_ref, wo_ref, bo_ref,
                o_ref, acc, cnt):
    i = pl.program_id(0)

    @pl.when(i == 0)
    def _():
        acc[...] = jnp.zeros_like(acc)
        cnt[...] = jnp.zeros_like(cnt)

    h = p_ref[0] + p_ref[1] + sl_ref[...] * z_ref[...] + b_ref[...]
    bt = batch_ref[0]
    gids = lax.broadcasted_iota(jnp.int32, (N_GRAPHS, RB), 0)
    onehot = jnp.where(bt == gids, 1.0, 0.0)
    acc[...] += lax.dot_general(
        onehot, h, (((1,), (0,)), ((), ())), preferred_element_type=jnp.float32)
    cnt[...] += jnp.broadcast_to(
        jnp.sum(onehot, axis=1, keepdims=True), (N_GRAPHS, D))

    @pl.when(i == N_NODES // RB - 1)
    def _():
        pooled = acc[...] / jnp.maximum(cnt[...], 1.0)
        o_ref[...] = lax.dot_general(
            pooled, wo_ref[...], (((1,), (1,)), ((), ())),
            preferred_element_type=jnp.float32) + bo_ref[...]


def _final_call(p, z, sl, b, batch4, Wo, bo):
    return pl.pallas_call(
        _final_body,
        grid=(N_NODES // RB,),
        in_specs=[
            pl.BlockSpec((NC, RB, D), lambda i: (0, i, 0)),
            pl.BlockSpec((RB, D), lambda i: (i, 0)),
            pl.BlockSpec((RB, 1), lambda i: (i, 0)),
            pl.BlockSpec((1, D), lambda i: (0, 0)),
            pl.BlockSpec((1, 1, RB), lambda i: (i, 0, 0)),
            pl.BlockSpec((N_CLASSES, D), lambda i: (0, 0)),
            pl.BlockSpec((1, N_CLASSES), lambda i: (0, 0)),
        ],
        out_specs=pl.BlockSpec((N_GRAPHS, N_CLASSES), lambda i: (0, 0)),
        out_shape=jax.ShapeDtypeStruct((N_GRAPHS, N_CLASSES), jnp.float32),
        scratch_shapes=[
            pltpu.VMEM((N_GRAPHS, D), jnp.float32),
            pltpu.VMEM((N_GRAPHS, D), jnp.float32),
        ],
    )(p, z, sl, b, batch4, Wo, bo)


def kernel(x, edge_index, batch, P_vec, W1, b1, W2, b2, W3, b3, Wo, bo):
    row = edge_index[0]
    col = edge_index[1]
    batch4 = batch.reshape(N_NODES // RB, 1, RB)

    deg_parts = _deg_kernel(col, P_vec).reshape(NW, N_NODES)
    dis, sl = _dis_call(deg_parts)
    dis = dis.reshape(N_NODES)
    sl = sl.reshape(N_NODES, 1)
    w = _w_kernel(row, col, P_vec, dis)

    z1 = _mm1_call(x, W1)
    p1 = _agg_kernel(z1, row, col, w)
    z2 = _layer_call(p1, z1, sl, b1.reshape(1, D), W2, relu=True)
    p2 = _agg_kernel(z2, row, col, w)
    z3 = _layer_call(p2, z2, sl, b2.reshape(1, D), W3, relu=True)
    p3 = _agg_kernel(z3, row, col, w)
    return _final_call(p3, z3, sl, b3.reshape(1, D), batch4, Wo,
                       bo.reshape(1, N_CLASSES))
